# baseline (speedup 1.0000x reference)
import functools

import jax
import jax.numpy as jnp
from jax import lax
from jax.experimental import pallas as pl
from jax.experimental.pallas import tpu as pltpu
from jax.experimental.pallas import tpu_sc as plsc

BATCH = 8
SEQ = 1024
RED = 1024
DMODEL = 512
NROWS = BATCH * SEQ

NUM_WORKERS = 32
ROWS_PER_WORKER = NROWS // NUM_WORKERS
CHUNK = 64
CACHE_ROWS = 120
LANES = 16


def _counts_body(d_ref, idx_ref, flag_ref):
    d = d_ref[...]
    hit = jnp.logical_or(d == 1, d == -1)
    c = jnp.sum(hit.astype(jnp.int32), axis=-1)
    idx_ref[...] = jnp.minimum(c, DMODEL - 1).reshape(1, 1, SEQ)
    ok = jnp.all(c.reshape(SEQ // ROWS_PER_WORKER, ROWS_PER_WORKER) < CACHE_ROWS,
                 axis=1)
    flag_ref[...] = jnp.broadcast_to(
        ok.astype(jnp.int32)[None, :, None], (1, SEQ // ROWS_PER_WORKER, LANES)
    )


def _centrality_counts(distances):
    return pl.pallas_call(
        _counts_body,
        grid=(BATCH,),
        in_specs=[pl.BlockSpec((1, SEQ, RED), lambda b: (b, 0, 0))],
        out_specs=[
            pl.BlockSpec((1, 1, SEQ), lambda b: (b, 0, 0)),
            pl.BlockSpec((1, SEQ // ROWS_PER_WORKER, LANES), lambda b: (b, 0, 0)),
        ],
        out_shape=[
            jax.ShapeDtypeStruct((BATCH, 1, SEQ), jnp.int32),
            jax.ShapeDtypeStruct((BATCH, SEQ // ROWS_PER_WORKER, LANES), jnp.int32),
        ],
    )(distances)


def _gather_body(table_hbm, idx_hbm, flag_hbm, out_hbm, idx_v, cache_v, fl_v,
                 rows0, rows1, gsem0, gsem1, osem0, osem1):
    sid = lax.axis_index("s")
    wid = sid * 2 + lax.axis_index("c")
    base = wid * ROWS_PER_WORKER
    nchunk = ROWS_PER_WORKER // CHUNK
    rows = (rows0, rows1)
    gsems = (gsem0, gsem1)
    osems = (osem0, osem1)
    iota = lax.iota(jnp.int32, LANES)

    pltpu.sync_copy(idx_hbm.at[pl.ds(base, ROWS_PER_WORKER)], idx_v)
    pltpu.sync_copy(flag_hbm.at[wid], fl_v)
    in_window = fl_v[...][0] == 1

    @pl.when(in_window)
    def _local_expand():
        pltpu.sync_copy(table_hbm.at[pl.ds(0, CACHE_ROWS)], cache_v)
        outs = [None] * nchunk
        for c in range(nchunk):
            b = c % 2
            if c >= 2:
                outs[c - 2].wait()

            def _row(i, carry, c=c, b=b):
                srow = jnp.full((LANES,), i, jnp.int32)
                trow = plsc.load_gather(idx_v, [c * CHUNK + srow])
                for j in range(DMODEL // LANES):
                    col = j * LANES + iota
                    w = plsc.load_gather(cache_v, [trow, col])
                    plsc.store_scatter(rows[b], [srow, col], w)
                return carry

            lax.fori_loop(0, CHUNK, _row, 0)
            outs[c] = pltpu.async_copy(
                rows[b], out_hbm.at[pl.ds(base + c * CHUNK, CHUNK)], osems[b]
            )
        outs[nchunk - 2].wait()
        outs[nchunk - 1].wait()

    @pl.when(jnp.logical_not(in_window))
    def _hbm_gather():
        gathers = [None] * nchunk
        outs = [None] * nchunk
        for c in range(nchunk):
            b = c % 2
            if c >= 2:
                outs[c - 2].wait()
            gathers[c] = pltpu.async_copy(
                table_hbm.at[idx_v.at[pl.ds(c * CHUNK, CHUNK)]], rows[b],
                gsems[b],
            )
            gathers[c].wait()
            outs[c] = pltpu.async_copy(
                rows[b], out_hbm.at[pl.ds(base + c * CHUNK, CHUNK)], osems[b]
            )
        outs[nchunk - 2].wait()
        outs[nchunk - 1].wait()


@functools.lru_cache(maxsize=1)
def _gather_rows():
    return pl.kernel(
        _gather_body,
        mesh=plsc.VectorSubcoreMesh(core_axis_name="c", subcore_axis_name="s"),
        compiler_params=pltpu.CompilerParams(needs_layout_passes=False),
        out_type=jax.ShapeDtypeStruct((NROWS, DMODEL), jnp.float32),
        scratch_types=[
            pltpu.VMEM((ROWS_PER_WORKER,), jnp.int32),
            pltpu.VMEM((CACHE_ROWS, DMODEL), jnp.float32),
            pltpu.VMEM((LANES,), jnp.int32),
            pltpu.VMEM((CHUNK, DMODEL), jnp.float32),
            pltpu.VMEM((CHUNK, DMODEL), jnp.float32),
            pltpu.SemaphoreType.DMA,
            pltpu.SemaphoreType.DMA,
            pltpu.SemaphoreType.DMA,
            pltpu.SemaphoreType.DMA,
        ],
    )


def kernel(distances, table):
    idx, flags = _centrality_counts(distances)
    rows = _gather_rows()(
        table, idx.reshape(NROWS), flags.reshape(NUM_WORKERS, LANES)
    )
    return rows.reshape(BATCH, SEQ, DMODEL)
